# baseline (speedup 1.0000x reference)
import jax
import jax.numpy as jnp
from jax import lax
from jax.experimental import pallas as pl
from jax.experimental.pallas import tpu as pltpu
from jax.experimental.pallas import tpu_sc as plsc

ENTITY_NUM = 1000000
RELATION_NUM = 1000
EMBED_DIM = 64
BATCH = 16384

NUM_CORES = 2
NUM_SUBCORES = 16
LANES = 16
DBLOCKS = EMBED_DIM // LANES

CHUNK = 512
RANGE_COLS = 31232
LIST_CAP = 6144
DUMP_ROW = 2 * BATCH
HT2_ROWS = DUMP_ROW + 8


def _gather_body(ent_hbm, tail_hbm, heads_hbm, tails_hbm, ht2_hbm,
                 hidx, tidx, el, bl, win_a, win_b, staging, bvec_v, tailv,
                 cnt_smem, sem_w, sem_sc):
    sid = lax.axis_index("s")
    cid = lax.axis_index("c")
    wid = sid * NUM_CORES + cid
    rstart = wid * RANGE_COLS
    nch = jnp.where(wid == 31, 63, 61)
    nhbm = jnp.where(wid == 31, 62, 61)

    pltpu.sync_copy(heads_hbm, hidx)
    pltpu.sync_copy(tails_hbm, tidx)
    pltpu.sync_copy(tail_hbm, tailv)
    cnt_smem[0] = 0
    lane = lax.broadcasted_iota(jnp.int32, (LANES,), 0)
    bvec_v[...] = jnp.full_like(lane, DUMP_ROW)
    dvs = [lane + j * LANES for j in range(DBLOCKS)]
    lane0 = lane == 0
    rend = rstart + nch * CHUNK

    def compact(idx_ref, tagoff):
        def vec(v, cnt):
            ev = idx_ref[pl.ds(v * LANES, LANES)]
            m = (ev >= rstart) & (ev < rend)
            plsc.store_compressed(el.at[pl.ds(cnt, LANES)], ev, mask=m)
            plsc.store_compressed(bl.at[pl.ds(cnt, LANES)],
                                  lane + (v * LANES + tagoff), mask=m)
            return cnt + plsc.all_reduce_population_count(m)[0]

        return vec

    cnt = lax.fori_loop(0, BATCH // LANES, compact(hidx, 0), 0)
    cnt = lax.fori_loop(0, BATCH // LANES, compact(tidx, BATCH), cnt)
    cnt_c = jnp.minimum(cnt, LIST_CAP)
    el[pl.ds(cnt_c, LANES)] = jnp.full_like(lane, 1 << 30)

    def enqueue_win(ch, buf):
        off = pl.multiple_of(rstart + ch * CHUNK, 128)
        pltpu.async_copy(ent_hbm.at[:, pl.ds(off, CHUNK)], buf, sem_w)

    def process(buf, ch, ev_fn, nv):
        def vec(v, carry):
            ev, bvv = ev_fn(v)
            m = ((ev - rstart) >> 9) == ch
            npop = plsc.all_reduce_population_count(m)[0]
            mi = m.astype(jnp.int32)

            @pl.when(npop > 0)
            def _():
                colv = (ev - rstart) & (CHUNK - 1)
                for k in range(LANES):
                    @pl.when(mi[k] != 0)
                    def _(k=k):
                        c = cnt_smem[0]
                        srow = c & 31
                        cc = jnp.full_like(lane, colv[k])
                        for j in range(DBLOCKS):
                            val = plsc.load_gather(buf, [dvs[j], cc])
                            staging[srow, pl.ds(j * LANES, LANES)] = val
                        plsc.store_scatter(
                            bvec_v, [jnp.full_like(lane, c & 15)],
                            jnp.full_like(lane, bvv[k]), mask=lane0)
                        cnt_smem[0] = c + 1

                        @pl.when((c & 15) == 15)
                        def _():
                            @pl.when(c > 16)
                            def _():
                                pltpu.make_async_copy(
                                    ht2_hbm.at[pl.ds(0, LANES)],
                                    staging.at[pl.ds(0, LANES)],
                                    sem_sc).wait()

                            bv = bvec_v[...]
                            pltpu.async_copy(staging.at[pl.ds(c & 16, LANES)],
                                             ht2_hbm.at[bv], sem_sc)
                            bvec_v[...] = jnp.full_like(lane, DUMP_ROW)

            return carry

        lax.fori_loop(0, nv, vec, 0)

    def list_ev(v):
        sl = pl.ds(v * LANES, LANES)
        return el[sl], bl[sl]

    def raw_h(v):
        return hidx[pl.ds(v * LANES, LANES)], lane + v * LANES

    def raw_t(v):
        return tidx[pl.ds(v * LANES, LANES)], lane + (v * LANES + BATCH)

    def chunk_loop(fast):
        def pair(pr, carry):
            for par, buf in ((0, win_a), (1, win_b)):
                ch = pr * 2 + par

                @pl.when(ch < nch)
                def _():
                    @pl.when(ch < nhbm)
                    def _():
                        pltpu.make_async_copy(
                            ent_hbm.at[:, pl.ds(0, CHUNK)], buf, sem_w).wait()

                    @pl.when(ch == 62)
                    def _():
                        def plant(d, c3):
                            for j in range(DBLOCKS):
                                sl = pl.ds(j * LANES, LANES)
                                buf[d, sl] = tailv[d, sl]
                            return c3

                        lax.fori_loop(0, EMBED_DIM, plant, 0)

                    if fast:
                        process(buf, ch, list_ev, (cnt_c + LANES) >> 4)
                    else:
                        process(buf, ch, raw_h, BATCH // LANES)
                        process(buf, ch, raw_t, BATCH // LANES)

                    @pl.when(ch + 2 < nhbm)
                    def _():
                        enqueue_win(ch + 2, buf)

            return carry

        lax.fori_loop(0, 32, pair, 0)

    enqueue_win(0, win_a)

    @pl.when(nhbm > 1)
    def _():
        enqueue_win(1, win_b)

    @pl.when(cnt <= LIST_CAP)
    def _():
        chunk_loop(True)

    @pl.when(cnt > LIST_CAP)
    def _():
        chunk_loop(False)

    c = cnt_smem[0]

    @pl.when((c & 15) != 0)
    def _():
        bv = bvec_v[...]
        pltpu.async_copy(staging.at[pl.ds(c & 16, LANES)],
                         ht2_hbm.at[bv], sem_sc)

    n_out = jnp.where(c >= 16, 1, 0) + jnp.where((c & 15) != 0, 1, 0)

    def drsc(i, c2):
        pltpu.make_async_copy(ht2_hbm.at[pl.ds(0, LANES)],
                              staging.at[pl.ds(0, LANES)], sem_sc).wait()
        return c2

    lax.fori_loop(0, n_out, drsc, 0)


_k1 = pl.kernel(
    _gather_body,
    out_type=jax.ShapeDtypeStruct((HT2_ROWS, 2 * EMBED_DIM), jnp.float32),
    mesh=plsc.VectorSubcoreMesh(
        core_axis_name="c", subcore_axis_name="s",
        num_cores=NUM_CORES, num_subcores=NUM_SUBCORES),
    scratch_types=[
        pltpu.VMEM((BATCH,), jnp.int32),
        pltpu.VMEM((BATCH,), jnp.int32),
        pltpu.VMEM((LIST_CAP + 2 * LANES,), jnp.int32),
        pltpu.VMEM((LIST_CAP + 2 * LANES,), jnp.int32),
        pltpu.VMEM((EMBED_DIM, CHUNK), jnp.float32),
        pltpu.VMEM((EMBED_DIM, CHUNK), jnp.float32),
        pltpu.VMEM((2 * LANES, 2 * EMBED_DIM), jnp.float32),
        pltpu.VMEM((LANES,), jnp.int32),
        pltpu.VMEM((EMBED_DIM, 2 * EMBED_DIM), jnp.float32),
        pltpu.SMEM((8,), jnp.int32),
        pltpu.SemaphoreType.DMA,
        pltpu.SemaphoreType.DMA,
    ],
    compiler_params=pltpu.CompilerParams(needs_layout_passes=False),
)


def _combine_body(ht2_hbm, rel_hbm, rels_hbm, out_hbm,
                  ridx, hbuf, tbuf, rbuf, obuf, sem_r):
    wid = lax.axis_index("s") * NUM_CORES + lax.axis_index("c")
    base = wid * 512

    pltpu.sync_copy(rels_hbm.at[pl.ds(base, 512)], ridx)
    lane = lax.broadcasted_iota(jnp.int32, (LANES,), 0)
    dvs = [lane + j * LANES for j in range(DBLOCKS)]

    for chunk in range(4):
        r0 = base + chunk * 128
        cr = pltpu.async_copy(
            rel_hbm.at[ridx.at[pl.ds(chunk * 128, 128)]], rbuf, sem_r)
        pltpu.sync_copy(ht2_hbm.at[pl.ds(r0, 128)], hbuf)
        pltpu.sync_copy(ht2_hbm.at[pl.ds(BATCH + r0, 128)], tbuf)
        cr.wait()

        def row(i, carry):
            ocol = jnp.full_like(lane, chunk * 128 + i)
            for j in range(DBLOCKS):
                sl = pl.ds(j * LANES, LANES)
                val = hbuf[i, sl] + rbuf[i, sl] - tbuf[i, sl]
                plsc.store_scatter(obuf, [dvs[j], ocol], val)
            return carry

        lax.fori_loop(0, 128, row, 0)

    pltpu.sync_copy(obuf, out_hbm.at[:, pl.ds(base, 512)])


_k2 = pl.kernel(
    _combine_body,
    out_type=jax.ShapeDtypeStruct((EMBED_DIM, BATCH), jnp.float32),
    mesh=plsc.VectorSubcoreMesh(
        core_axis_name="c", subcore_axis_name="s",
        num_cores=NUM_CORES, num_subcores=NUM_SUBCORES),
    scratch_types=[
        pltpu.VMEM((512,), jnp.int32),
        pltpu.VMEM((128, 2 * EMBED_DIM), jnp.float32),
        pltpu.VMEM((128, 2 * EMBED_DIM), jnp.float32),
        pltpu.VMEM((128, 2 * EMBED_DIM), jnp.float32),
        pltpu.VMEM((EMBED_DIM, 512), jnp.float32),
        pltpu.SemaphoreType.DMA,
    ],
    compiler_params=pltpu.CompilerParams(needs_layout_passes=False),
)


@jax.jit
def kernel(entity_emb, relation_emb, heads, relations, tails):
    ent_t = entity_emb.T
    rel_pad = jnp.pad(relation_emb, ((0, 0), (0, EMBED_DIM)))
    tail_pad = jnp.pad(entity_emb[999936:].T, ((0, 0), (0, EMBED_DIM)))
    ht2 = _k1(ent_t, tail_pad, heads.astype(jnp.int32), tails.astype(jnp.int32))
    out_t = _k2(ht2, rel_pad, relations.astype(jnp.int32))
    return out_t.T
